# Initial kernel scaffold; baseline (speedup 1.0000x reference)
import jax
import jax.numpy as jnp
from jax.experimental import pallas as pl


def kernel(input_ids, token_type_ids, token_weight, pos_weight, type_weight, ln_weight, ln_bias):
    raise NotImplementedError("write your pallas kernel here")


# speedup vs baseline: 2.1030x; 2.1030x over previous
import functools

import jax
import jax.numpy as jnp
from jax import lax
from jax.experimental import pallas as pl
from jax.experimental.pallas import tpu as pltpu
from jax.experimental.pallas import tpu_sc as plsc

VOCAB = 30522
HID = 768
EPS = 1e-5

_NC = 2
_NS = 16
_NW = _NC * _NS
_LANES = 16
_CHUNKS = HID // _LANES


def _sc_body(tok_hbm, pos_hbm, type_hbm, w_hbm, b_hbm, ids_hbm, tt_hbm,
             out_hbm, idx_v, tt_v, tok_rows, pos_rows, type_v, w_v, b_v, sem):
    wid = lax.axis_index("s") * _NC + lax.axis_index("c")
    bpw = 2048 // _NW
    base = wid * bpw

    pltpu.sync_copy(ids_hbm.at[pl.ds(base, bpw)], idx_v)
    gather = pltpu.async_copy(tok_hbm.at[idx_v], tok_rows, sem)
    pltpu.sync_copy(tt_hbm.at[pl.ds(base, bpw)], tt_v)
    pltpu.sync_copy(pos_hbm.at[pl.ds(base, bpw)], pos_rows)
    pltpu.sync_copy(type_hbm, type_v)
    pltpu.sync_copy(w_hbm, w_v)
    pltpu.sync_copy(b_hbm, b_v)
    gather.wait()

    inv_h = jnp.float32(1.0 / HID)

    def per_token(i, carry):
        lane_i = jnp.full((_LANES,), i, jnp.int32)
        ttf = plsc.load_gather(tt_v, [lane_i]).astype(jnp.float32)
        s = jnp.zeros((_LANES,), jnp.float32)
        ss = jnp.zeros((_LANES,), jnp.float32)
        for j in range(_CHUNKS):
            sl = pl.ds(j * _LANES, _LANES)
            t0 = type_v[0, sl]
            t1 = type_v[1, sl]
            e = tok_rows[i, sl] + pos_rows[i, sl] + t0 + ttf * (t1 - t0)
            tok_rows[i, sl] = e
            s = s + e
            ss = ss + e * e
        mean = jnp.sum(s) * inv_h
        var = jnp.sum(ss) * inv_h - mean * mean
        x = var + jnp.float32(EPS)
        xi = lax.bitcast_convert_type(x, jnp.int32)
        yi = jnp.int32(0x5F3759DF) - lax.shift_right_logical(xi, 1)
        y = lax.bitcast_convert_type(yi, jnp.float32)
        half_x = jnp.float32(0.5) * x
        for _ in range(3):
            y = y * (jnp.float32(1.5) - half_x * y * y)
        for j in range(_CHUNKS):
            sl = pl.ds(j * _LANES, _LANES)
            e = tok_rows[i, sl]
            tok_rows[i, sl] = (e - mean) * y * w_v[sl] + b_v[sl]
        return carry

    lax.fori_loop(0, bpw, per_token, 0)

    pltpu.sync_copy(tok_rows, out_hbm.at[pl.ds(base, bpw)])


@jax.jit
def _bert_embed_sc(token_weight, pos_weight, type_weight, ln_weight, ln_bias,
                   ids, tt):
    bpw = 2048 // _NW
    run = functools.partial(
        pl.kernel,
        mesh=plsc.VectorSubcoreMesh(core_axis_name="c", subcore_axis_name="s"),
        out_type=jax.ShapeDtypeStruct((2048, HID), jnp.float32),
        scratch_types=[
            pltpu.VMEM((bpw,), jnp.int32),
            pltpu.VMEM((bpw,), jnp.int32),
            pltpu.VMEM((bpw, HID), jnp.float32),
            pltpu.VMEM((bpw, HID), jnp.float32),
            pltpu.VMEM((2, HID), jnp.float32),
            pltpu.VMEM((HID,), jnp.float32),
            pltpu.VMEM((HID,), jnp.float32),
            pltpu.SemaphoreType.DMA,
        ],
        compiler_params=pltpu.CompilerParams(needs_layout_passes=False),
    )(_sc_body)
    return run(token_weight, pos_weight, type_weight, ln_weight, ln_bias,
               ids, tt)


def kernel(input_ids, token_type_ids, token_weight, pos_weight, type_weight,
           ln_weight, ln_bias):
    b, s = input_ids.shape
    ids = input_ids.reshape(s).astype(jnp.int32)
    tt = token_type_ids.reshape(s).astype(jnp.int32)
    out = _bert_embed_sc(token_weight, pos_weight, type_weight,
                         ln_weight, ln_bias, ids, tt)
    return out.reshape(b, s, HID)
